# baseline (speedup 1.0000x reference)
import jax
import jax.numpy as jnp
from jax import lax
from jax.experimental import pallas as pl
from jax.experimental.pallas import tpu as pltpu
from jax.experimental.pallas import tpu_sc as plsc

_ROW = 32
_NW = 32
_CH = 512
_G = _CH // 16


def _batcher_pairs(n):
    pairs = []

    def merge(lo, m, r):
        step = r * 2
        if step < m:
            merge(lo, m, step)
            merge(lo + r, m, step)
            for i in range(lo + r, lo + m - r, step):
                pairs.append((i, i + r))
        else:
            pairs.append((lo, lo + r))

    def sort(lo, m):
        if m > 1:
            k = m // 2
            sort(lo, k)
            sort(lo + k, k)
            merge(lo, m, 1)

    sort(0, n)
    return pairs


_PAIRS = _batcher_pairs(_ROW)


def _sc_body(x_hbm, o_hbm, buf, obuf):
    n = o_hbm.shape[0]
    rpw = n // _NW
    wid = lax.axis_index("s") * 2 + lax.axis_index("c")
    base_row = wid * rpw
    lane = lax.iota(jnp.int32, 16)

    def chunk(c, _):
        row0 = base_row + c * _CH
        pltpu.sync_copy(x_hbm.at[pl.ds(row0, _CH), :], buf)

        def group(g, _):
            rows = lane + g * 16
            vs = [plsc.load_gather(buf, [rows, jnp.full((16,), k, jnp.int32)])
                  for k in range(_ROW)]
            for (i, j) in _PAIRS:
                a, b = vs[i], vs[j]
                vs[i] = jnp.minimum(a, b)
                vs[j] = jnp.maximum(a, b)
            eqs = [vs[k] == vs[k - 1] for k in range(1, _ROW)]
            anydup = eqs[0]
            for e in eqs[1:]:
                anydup = anydup | e
            has_dup = jnp.max(anydup.astype(jnp.int32))

            def with_scan():
                run = jnp.ones((16,), jnp.int32)
                best = run
                bestv = vs[0]
                for k in range(1, _ROW):
                    run = run * eqs[k - 1].astype(jnp.int32) + 1
                    bt = run > best
                    best = jnp.maximum(run, best)
                    bestv = jnp.where(bt, vs[k], bestv)
                return bestv

            bestv = lax.cond(has_dup > 0, with_scan, lambda: vs[0])
            obuf[pl.ds(g * 16, 16)] = bestv
            return 0

        lax.fori_loop(0, _G, group, 0)
        pltpu.sync_copy(obuf, o_hbm.at[pl.ds(row0, _CH)])
        return 0

    lax.fori_loop(0, rpw // _CH, chunk, 0)


def kernel(x):
    n = x.shape[0]
    out = pl.kernel(
        _sc_body,
        out_type=jax.ShapeDtypeStruct((n,), jnp.float32),
        mesh=plsc.VectorSubcoreMesh(core_axis_name="c", subcore_axis_name="s"),
        scratch_types=[
            pltpu.VMEM((_CH, _ROW), jnp.float32),
            pltpu.VMEM((_CH,), jnp.float32),
        ],
        compiler_params=pltpu.CompilerParams(needs_layout_passes=False),
    )(x)
    return out
